# Initial kernel scaffold; baseline (speedup 1.0000x reference)
import jax
import jax.numpy as jnp
from jax.experimental import pallas as pl


def kernel(x, edge_index, W1, b1, W2, b2, W3, b3, gamma2, beta2, gamma3, beta3):
    raise NotImplementedError("write your pallas kernel here")


# speedup vs baseline: 1.1100x; 1.1100x over previous
import functools

import jax
import jax.numpy as jnp
from jax.experimental import pallas as pl
from jax.experimental.pallas import tpu as pltpu

V_NODES = 10000
KCHEB = 6
NB = 4
EPS = 1e-5


def _cheb_matmul_kernel(x_ref, w_ref, bias_ref, o_ref, acc_ref, *, nk, nb, relu):
    k = pl.program_id(1)

    @pl.when(k == 0)
    def _():
        acc_ref[...] = jnp.zeros_like(acc_ref)

    xblk = x_ref[0]
    wblk = w_ref[0]
    c = wblk.shape[0]
    cout = wblk.shape[1]
    for b in range(nb):
        acc_ref[:, b * cout:(b + 1) * cout] += jnp.dot(
            xblk[:, b * c:(b + 1) * c], wblk, preferred_element_type=jnp.float32)

    @pl.when(k == nk - 1)
    def _():
        bias_row = bias_ref[...]
        res = acc_ref[...] + bias_row[None, :]
        if relu:
            res = jnp.maximum(res, 0.0)
        o_ref[...] = res


def _cheb_matmul(xk, w, bias, relu=True, vt=1000):
    nk, v, ncb = xk.shape
    c, cout = w.shape[1], w.shape[2]
    bias_t = jnp.tile(bias, NB)
    grid = (v // vt, nk)
    return pl.pallas_call(
        functools.partial(_cheb_matmul_kernel, nk=nk, nb=NB, relu=relu),
        grid=grid,
        in_specs=[
            pl.BlockSpec((1, vt, ncb), lambda i, k: (k, i, 0)),
            pl.BlockSpec((1, c, cout), lambda i, k: (k, 0, 0)),
            pl.BlockSpec((NB * cout,), lambda i, k: (0,)),
        ],
        out_specs=pl.BlockSpec((vt, NB * cout), lambda i, k: (i, 0)),
        out_shape=jax.ShapeDtypeStruct((v, NB * cout), jnp.float32),
        scratch_shapes=[pltpu.VMEM((vt, NB * cout), jnp.float32)],
    )(xk, w, bias_t)


def _apply_adj(z, src, dst):
    return jax.ops.segment_sum(z[src], dst, num_segments=V_NODES)


def _cheb_stack(h, src, dst, u):
    un = u[:, None]
    xs = [h]
    x0 = h
    x1 = -un * _apply_adj(un * h, src, dst)
    xs.append(x1)
    for _ in range(2, KCHEB):
        x2 = -2.0 * un * _apply_adj(un * x1, src, dst) - x0
        xs.append(x2)
        x0, x1 = x1, x2
    return jnp.stack(xs, axis=0)


def _bn(h, gamma, beta):
    v = h.shape[0]
    hb = h.reshape(v, NB, -1)
    mean = jnp.mean(hb, axis=(0, 1))
    var = jnp.var(hb, axis=(0, 1))
    out = gamma * (hb - mean) / jnp.sqrt(var + EPS) + beta
    return out.reshape(v, -1)


def kernel(x, edge_index, W1, b1, W2, b2, W3, b3, gamma2, beta2, gamma3, beta3):
    src = edge_index[0]
    dst = edge_index[1]
    ones = jnp.ones((src.shape[0],), dtype=jnp.float32)
    deg = jax.ops.segment_sum(ones, dst, num_segments=V_NODES)
    u = 1.0 / jnp.sqrt(jnp.clip(deg, 1.0, None))

    h = jnp.transpose(x, (2, 0, 1)).reshape(V_NODES, -1)

    xk = _cheb_stack(h, src, dst, u)
    h = _cheb_matmul(xk, W1, b1, relu=True)
    h = _bn(h, gamma2, beta2)

    xk = _cheb_stack(h, src, dst, u)
    h = _cheb_matmul(xk, W2, b2, relu=True)
    h = _bn(h, gamma3, beta3)

    xk = _cheb_stack(h, src, dst, u)
    h = _cheb_matmul(xk, W3, b3, relu=True)

    pooled = jnp.mean(h, axis=0).reshape(NB, -1)
    return jax.nn.log_softmax(pooled, axis=1)


# speedup vs baseline: 3.4823x; 3.1372x over previous
import functools

import jax
import jax.numpy as jnp
from jax import lax
from jax.experimental import pallas as pl
from jax.experimental.pallas import tpu as pltpu
from jax.experimental.pallas import tpu_sc as plsc

V_NODES = 10000
KCHEB = 6
NB = 4
EPS = 1e-5

E_EDGES = 160000
EW = 64
NWIN = E_EDGES // EW
NSC, NSUB = 2, 16
WIN_PS = 160
NWINP = WIN_PS * NSUB
VPAD = 10016
ZROWS = 160
WIN_H = 80

_SC_MESH = plsc.VectorSubcoreMesh(core_axis_name="c", subcore_axis_name="s")


def _spmv_body(nt, ct, z_hbm, src_hbm, dst_hbm, zc_hbm, y_hbm,
               rows_v, sidx_v, didx_v, acc_sh, sem0, sem1):
    core = lax.axis_index("c")
    sub = lax.axis_index("s")
    sems = (sem0, sem1)

    for t in range(nt):
        @pl.when(core == (t % NSC))
        def _process(t=t):
            zt = z_hbm.at[t]

            @pl.when(sub < NSUB - 1)
            def _():
                for j in range(4):
                    pltpu.sync_copy(
                        zc_hbm, acc_sh.at[pl.ds(sub * 640 + j * ZROWS, ZROWS)])

            @pl.when(sub == NSUB - 1)
            def _():
                pltpu.sync_copy(zc_hbm, acc_sh.at[pl.ds(9600, ZROWS)])
                pltpu.sync_copy(zc_hbm, acc_sh.at[pl.ds(9760, ZROWS)])
                pltpu.sync_copy(zc_hbm.at[pl.ds(0, 80)],
                                acc_sh.at[pl.ds(9920, 80)])

            plsc.subcore_barrier()

            def fire(w, b):
                pltpu.async_copy(zt.at[sidx_v.at[w]], rows_v.at[b], sems[b])

            def drain(w, b):
                pltpu.make_async_copy(zt.at[pl.ds(0, EW)], rows_v.at[b],
                                      sems[b]).wait()
                pltpu.sync_copy(rows_v.at[b],
                                acc_sh.at[didx_v.at[w]], add=True)

            for h in range(WIN_PS // WIN_H):
                pltpu.sync_copy(
                    src_hbm.at[pl.ds(sub * WIN_PS + h * WIN_H, WIN_H)],
                    sidx_v)
                pltpu.sync_copy(
                    dst_hbm.at[pl.ds(sub * WIN_PS + h * WIN_H, WIN_H)],
                    didx_v)
                fire(0, 0)

                @pl.loop(0, (WIN_H - 2) // 2)
                def _(i):
                    w = 2 * i
                    fire(w + 1, 1)
                    drain(w, 0)
                    fire(w + 2, 0)
                    drain(w + 1, 1)

                fire(WIN_H - 1, 1)
                drain(WIN_H - 2, 0)
                drain(WIN_H - 1, 1)

            plsc.subcore_barrier()

            @pl.when(sub < NSUB - 1)
            def _():
                pltpu.sync_copy(
                    acc_sh.at[pl.ds(sub * 640, 640)],
                    y_hbm.at[pl.ds(t * V_NODES + sub * 640, 640)])

            @pl.when(sub == NSUB - 1)
            def _():
                pltpu.sync_copy(
                    acc_sh.at[pl.ds(9600, 400)],
                    y_hbm.at[pl.ds(t * V_NODES + 9600, 400)])

            plsc.subcore_barrier()


@functools.lru_cache(maxsize=None)
def _make_spmv(nt, ct):
    body = functools.partial(_spmv_body, nt, ct)
    return pl.kernel(
        body,
        out_type=jax.ShapeDtypeStruct((nt * V_NODES, ct), jnp.float32),
        mesh=_SC_MESH,
        scratch_types=[
            pltpu.VMEM((2, EW, ct), jnp.float32),
            pltpu.VMEM((WIN_H, EW), jnp.int32),
            pltpu.VMEM((WIN_H, EW), jnp.int32),
            pltpu.VMEM_SHARED((VPAD, ct), jnp.float32),
            pltpu.SemaphoreType.DMA,
            pltpu.SemaphoreType.DMA,
        ],
    )


def _adj_apply(z_tm, src2d, dst2d):
    nt, v, ct = z_tm.shape
    zc = jnp.zeros((ZROWS, ct), jnp.float32)
    y = _make_spmv(nt, ct)(z_tm, src2d, dst2d, zc)
    return y.reshape(nt, v, ct)


def _pad_windows(idx, fill):
    npad = NWINP - NWIN
    pad = jnp.full((npad, EW), fill, jnp.int32)
    return jnp.concatenate([idx.reshape(NWIN, EW), pad])


def _mm_kernel(*refs, nk, nt_in, tpb, cout, relu):
    x_refs = refs[:nk]
    w_ref, bias_ref, o_ref = refs[nk], refs[nk + 1], refs[nk + 2]
    ct = x_refs[0].shape[2]
    hpb = cout // ct if cout >= ct else 1
    for b in range(NB):
        acc = None
        for k in range(nk):
            if tpb == 1:
                xb = x_refs[k][b]
            else:
                xb = jnp.concatenate(
                    [x_refs[k][b * tpb + j] for j in range(tpb)], axis=1)
            d = jnp.dot(xb, w_ref[k], preferred_element_type=jnp.float32)
            acc = d if acc is None else acc + d
        res = acc + bias_ref[...].reshape(-1)[None, :]
        if relu:
            res = jnp.maximum(res, 0.0)
        if cout >= ct:
            for j in range(hpb):
                o_ref[b * hpb + j] = res[:, j * ct:(j + 1) * ct]
        else:
            o_ref[b] = res


def _cheb_matmul(xs, w, bias, relu=True, vt=400):
    nk = len(xs)
    nt_in, v, ct = xs[0].shape
    c, cout = w.shape[1], w.shape[2]
    tpb = nt_in // NB
    nt_out = (NB * cout) // ct if cout >= ct else NB
    ct_out = ct if cout >= ct else cout
    grid = (v // vt,)
    bias2 = bias.reshape(-1, ct_out)
    return pl.pallas_call(
        functools.partial(_mm_kernel, nk=nk, nt_in=nt_in, tpb=tpb,
                          cout=cout, relu=relu),
        grid=grid,
        in_specs=[pl.BlockSpec((nt_in, vt, ct), lambda i: (0, i, 0))] * nk
        + [
            pl.BlockSpec(w.shape, lambda i: (0, 0, 0)),
            pl.BlockSpec(bias2.shape, lambda i: (0, 0)),
        ],
        out_specs=pl.BlockSpec((nt_out, vt, ct_out), lambda i: (0, i, 0)),
        out_shape=jax.ShapeDtypeStruct((nt_out, v, ct_out), jnp.float32),
    )(*xs, w, bias2)


def _y_matmul_kernel(x_ref, w_ref, o_ref, *, nk, tpb, coutp):
    for k in range(nk):
        parts = []
        for b in range(NB):
            xb = jnp.concatenate(
                [x_ref[b * tpb + j] for j in range(tpb)], axis=1)
            parts.append(jnp.dot(xb, w_ref[k],
                                 preferred_element_type=jnp.float32))
        o_ref[k] = jnp.concatenate(parts, axis=1)


def _y_matmul(h_tm, w, vt=400):
    nt, v, ct = h_tm.shape
    nk, c, coutp = w.shape
    tpb = nt // NB
    grid = (v // vt,)
    return pl.pallas_call(
        functools.partial(_y_matmul_kernel, nk=nk, tpb=tpb, coutp=coutp),
        grid=grid,
        in_specs=[
            pl.BlockSpec((nt, vt, ct), lambda i: (0, i, 0)),
            pl.BlockSpec(w.shape, lambda i: (0, 0, 0)),
        ],
        out_specs=pl.BlockSpec((nk, vt, NB * coutp), lambda i: (0, i, 0)),
        out_shape=jax.ShapeDtypeStruct((nk, v, NB * coutp), jnp.float32),
    )(h_tm, w)


def _cheb_xs(h_tm, srcw, dst2d, u_col):
    xs = [h_tm]
    x0 = h_tm
    x1 = -u_col * _adj_apply(u_col * h_tm, srcw, dst2d)
    xs.append(x1)
    for _ in range(2, KCHEB):
        x2 = -2.0 * u_col * _adj_apply(u_col * x1, srcw, dst2d) - x0
        xs.append(x2)
        x0, x1 = x1, x2
    return xs


def _bn_tm(h_tm, gamma, beta):
    nt, v, ct = h_tm.shape
    g = h_tm.reshape(NB, nt // NB, v, ct)
    mean = jnp.mean(g, axis=(0, 2), keepdims=True)
    var = jnp.var(g, axis=(0, 2), keepdims=True)
    gm = gamma.reshape(1, nt // NB, 1, ct)
    bt = beta.reshape(1, nt // NB, 1, ct)
    out = gm * (g - mean) * jax.lax.rsqrt(var + EPS) + bt
    return out.reshape(nt, v, ct)


def kernel(x, edge_index, W1, b1, W2, b2, W3, b3, gamma2, beta2, gamma3, beta3):
    src = edge_index[0]
    dst = edge_index[1]
    src2d = _pad_windows(src, 0)
    dst2d = _pad_windows(dst, V_NODES)

    ones128 = jnp.ones((1, V_NODES, 128), jnp.float32)
    deg = _adj_apply(ones128, src2d, dst2d)[0, :, 0]
    u = 1.0 / jnp.sqrt(jnp.clip(deg, 1.0, None))
    u_col = u[None, :, None]

    h = jnp.transpose(x, (0, 2, 1))
    xs = _cheb_xs(h, src2d, dst2d, u_col)
    h = _cheb_matmul(xs, W1, b1, relu=True)
    h = _bn_tm(h, gamma2, beta2)

    xs = _cheb_xs(h, src2d, dst2d, u_col)
    h = _cheb_matmul(xs, W2, b2, relu=True)
    h = _bn_tm(h, gamma3, beta3)

    coutp = 32
    w3p = jnp.pad(W3, ((0, 0), (0, 0), (0, coutp - W3.shape[2])))
    ys = _y_matmul(h, w3p)
    u1 = u[:, None]

    def lz(z):
        return -u1 * _adj_apply((u1 * z)[None], src2d, dst2d)[0]

    bk2 = ys[5]
    bk1 = ys[4] + 2.0 * lz(bk2)
    for k in (3, 2, 1):
        bk0 = ys[k] + 2.0 * lz(bk1) - bk2
        bk2, bk1 = bk1, bk0
    out3 = ys[0] + lz(bk1) - bk2

    b3p = jnp.pad(b3, (0, coutp - b3.shape[0]))
    h3 = jnp.maximum(out3 + jnp.tile(b3p, NB)[None, :], 0.0)
    pooled = jnp.mean(h3, axis=0).reshape(NB, coutp)[:, :W3.shape[2]]
    return jax.nn.log_softmax(pooled, axis=1)


# speedup vs baseline: 3.6457x; 1.0469x over previous
import functools

import jax
import jax.numpy as jnp
from jax import lax
from jax.experimental import pallas as pl
from jax.experimental.pallas import tpu as pltpu
from jax.experimental.pallas import tpu_sc as plsc

V_NODES = 10000
KCHEB = 6
NB = 4
EPS = 1e-5

E_EDGES = 160000
EW = 128
NWIN = E_EDGES // EW
NSC, NSUB = 2, 16
WIN_PS = 80
NWINP = WIN_PS * NSUB
VPAD = 10016
ZROWS = 160
WIN_H = 40

_SC_MESH = plsc.VectorSubcoreMesh(core_axis_name="c", subcore_axis_name="s")


def _spmv_body(nt, ct, z_hbm, src_hbm, dst_hbm, zc_hbm, y_hbm,
               rows_v, sidx_v, didx_v, acc_sh, sem0, sem1):
    core = lax.axis_index("c")
    sub = lax.axis_index("s")
    sems = (sem0, sem1)

    for t in range(nt):
        @pl.when(core == (t % NSC))
        def _process(t=t):
            zt = z_hbm.at[t]

            @pl.when(sub < NSUB - 1)
            def _():
                for j in range(4):
                    pltpu.sync_copy(
                        zc_hbm, acc_sh.at[pl.ds(sub * 640 + j * ZROWS, ZROWS)])

            @pl.when(sub == NSUB - 1)
            def _():
                pltpu.sync_copy(zc_hbm, acc_sh.at[pl.ds(9600, ZROWS)])
                pltpu.sync_copy(zc_hbm, acc_sh.at[pl.ds(9760, ZROWS)])
                pltpu.sync_copy(zc_hbm.at[pl.ds(0, 80)],
                                acc_sh.at[pl.ds(9920, 80)])

            plsc.subcore_barrier()

            def fire(w, b):
                pltpu.async_copy(zt.at[sidx_v.at[w]], rows_v.at[b], sems[b])

            def drain(w, b):
                pltpu.make_async_copy(zt.at[pl.ds(0, EW)], rows_v.at[b],
                                      sems[b]).wait()
                pltpu.sync_copy(rows_v.at[b],
                                acc_sh.at[didx_v.at[w]], add=True)

            for h in range(WIN_PS // WIN_H):
                pltpu.sync_copy(
                    src_hbm.at[pl.ds(sub * WIN_PS + h * WIN_H, WIN_H)],
                    sidx_v)
                pltpu.sync_copy(
                    dst_hbm.at[pl.ds(sub * WIN_PS + h * WIN_H, WIN_H)],
                    didx_v)
                fire(0, 0)

                @pl.loop(0, (WIN_H - 2) // 2)
                def _(i):
                    w = 2 * i
                    fire(w + 1, 1)
                    drain(w, 0)
                    fire(w + 2, 0)
                    drain(w + 1, 1)

                fire(WIN_H - 1, 1)
                drain(WIN_H - 2, 0)
                drain(WIN_H - 1, 1)

            plsc.subcore_barrier()

            @pl.when(sub < NSUB - 1)
            def _():
                pltpu.sync_copy(
                    acc_sh.at[pl.ds(sub * 640, 640)],
                    y_hbm.at[pl.ds(t * V_NODES + sub * 640, 640)])

            @pl.when(sub == NSUB - 1)
            def _():
                pltpu.sync_copy(
                    acc_sh.at[pl.ds(9600, 400)],
                    y_hbm.at[pl.ds(t * V_NODES + 9600, 400)])

            plsc.subcore_barrier()


@functools.lru_cache(maxsize=None)
def _make_spmv(nt, ct):
    body = functools.partial(_spmv_body, nt, ct)
    return pl.kernel(
        body,
        out_type=jax.ShapeDtypeStruct((nt * V_NODES, ct), jnp.float32),
        mesh=_SC_MESH,
        scratch_types=[
            pltpu.VMEM((2, EW, ct), jnp.float32),
            pltpu.VMEM((WIN_H, EW), jnp.int32),
            pltpu.VMEM((WIN_H, EW), jnp.int32),
            pltpu.VMEM_SHARED((VPAD, ct), jnp.float32),
            pltpu.SemaphoreType.DMA,
            pltpu.SemaphoreType.DMA,
        ],
    )


def _adj_apply(z_tm, src2d, dst2d):
    nt, v, ct = z_tm.shape
    zc = jnp.zeros((ZROWS, ct), jnp.float32)
    y = _make_spmv(nt, ct)(z_tm, src2d, dst2d, zc)
    return y.reshape(nt, v, ct)


def _pad_windows(idx, fill):
    npad = NWINP - NWIN
    pad = jnp.full((npad, EW), fill, jnp.int32)
    return jnp.concatenate([idx.reshape(NWIN, EW), pad])


def _mm_kernel(*refs, nk, nt_in, tpb, cout, relu):
    x_refs = refs[:nk]
    w_ref, bias_ref, o_ref = refs[nk], refs[nk + 1], refs[nk + 2]
    ct = x_refs[0].shape[2]
    hpb = cout // ct if cout >= ct else 1
    for b in range(NB):
        acc = None
        for k in range(nk):
            if tpb == 1:
                xb = x_refs[k][b]
            else:
                xb = jnp.concatenate(
                    [x_refs[k][b * tpb + j] for j in range(tpb)], axis=1)
            d = jnp.dot(xb, w_ref[k], preferred_element_type=jnp.float32)
            acc = d if acc is None else acc + d
        res = acc + bias_ref[...].reshape(-1)[None, :]
        if relu:
            res = jnp.maximum(res, 0.0)
        if cout >= ct:
            for j in range(hpb):
                o_ref[b * hpb + j] = res[:, j * ct:(j + 1) * ct]
        else:
            o_ref[b] = res


def _cheb_matmul(xs, w, bias, relu=True, vt=400):
    nk = len(xs)
    nt_in, v, ct = xs[0].shape
    c, cout = w.shape[1], w.shape[2]
    tpb = nt_in // NB
    nt_out = (NB * cout) // ct if cout >= ct else NB
    ct_out = ct if cout >= ct else cout
    grid = (v // vt,)
    bias2 = bias.reshape(-1, ct_out)
    return pl.pallas_call(
        functools.partial(_mm_kernel, nk=nk, nt_in=nt_in, tpb=tpb,
                          cout=cout, relu=relu),
        grid=grid,
        in_specs=[pl.BlockSpec((nt_in, vt, ct), lambda i: (0, i, 0))] * nk
        + [
            pl.BlockSpec(w.shape, lambda i: (0, 0, 0)),
            pl.BlockSpec(bias2.shape, lambda i: (0, 0)),
        ],
        out_specs=pl.BlockSpec((nt_out, vt, ct_out), lambda i: (0, i, 0)),
        out_shape=jax.ShapeDtypeStruct((nt_out, v, ct_out), jnp.float32),
    )(*xs, w, bias2)


def _y_matmul_kernel(x_ref, w_ref, o_ref, *, nk, tpb, coutp):
    for k in range(nk):
        parts = []
        for b in range(NB):
            xb = jnp.concatenate(
                [x_ref[b * tpb + j] for j in range(tpb)], axis=1)
            parts.append(jnp.dot(xb, w_ref[k],
                                 preferred_element_type=jnp.float32))
        o_ref[k] = jnp.concatenate(parts, axis=1)


def _y_matmul(h_tm, w, vt=400):
    nt, v, ct = h_tm.shape
    nk, c, coutp = w.shape
    tpb = nt // NB
    grid = (v // vt,)
    return pl.pallas_call(
        functools.partial(_y_matmul_kernel, nk=nk, tpb=tpb, coutp=coutp),
        grid=grid,
        in_specs=[
            pl.BlockSpec((nt, vt, ct), lambda i: (0, i, 0)),
            pl.BlockSpec(w.shape, lambda i: (0, 0, 0)),
        ],
        out_specs=pl.BlockSpec((nk, vt, NB * coutp), lambda i: (0, i, 0)),
        out_shape=jax.ShapeDtypeStruct((nk, v, NB * coutp), jnp.float32),
    )(h_tm, w)


def _cheb_xs(h_tm, srcw, dst2d, u_col):
    xs = [h_tm]
    x0 = h_tm
    x1 = -u_col * _adj_apply(u_col * h_tm, srcw, dst2d)
    xs.append(x1)
    for _ in range(2, KCHEB):
        x2 = -2.0 * u_col * _adj_apply(u_col * x1, srcw, dst2d) - x0
        xs.append(x2)
        x0, x1 = x1, x2
    return xs


def _bn_tm(h_tm, gamma, beta):
    nt, v, ct = h_tm.shape
    g = h_tm.reshape(NB, nt // NB, v, ct)
    mean = jnp.mean(g, axis=(0, 2), keepdims=True)
    var = jnp.var(g, axis=(0, 2), keepdims=True)
    gm = gamma.reshape(1, nt // NB, 1, ct)
    bt = beta.reshape(1, nt // NB, 1, ct)
    out = gm * (g - mean) * jax.lax.rsqrt(var + EPS) + bt
    return out.reshape(nt, v, ct)


def kernel(x, edge_index, W1, b1, W2, b2, W3, b3, gamma2, beta2, gamma3, beta3):
    src = edge_index[0]
    dst = edge_index[1]
    src2d = _pad_windows(src, 0)
    dst2d = _pad_windows(dst, V_NODES)

    ones128 = jnp.ones((1, V_NODES, 128), jnp.float32)
    deg = _adj_apply(ones128, src2d, dst2d)[0, :, 0]
    u = 1.0 / jnp.sqrt(jnp.clip(deg, 1.0, None))
    u_col = u[None, :, None]

    h = jnp.transpose(x, (0, 2, 1))
    xs = _cheb_xs(h, src2d, dst2d, u_col)
    h = _cheb_matmul(xs, W1, b1, relu=True)
    h = _bn_tm(h, gamma2, beta2)

    xs = _cheb_xs(h, src2d, dst2d, u_col)
    h = _cheb_matmul(xs, W2, b2, relu=True)
    h = _bn_tm(h, gamma3, beta3)

    coutp = 32
    w3p = jnp.pad(W3, ((0, 0), (0, 0), (0, coutp - W3.shape[2])))
    ys = _y_matmul(h, w3p)
    u1 = u[:, None]

    def lz(z):
        return -u1 * _adj_apply((u1 * z)[None], src2d, dst2d)[0]

    bk2 = ys[5]
    bk1 = ys[4] + 2.0 * lz(bk2)
    for k in (3, 2, 1):
        bk0 = ys[k] + 2.0 * lz(bk1) - bk2
        bk2, bk1 = bk1, bk0
    out3 = ys[0] + lz(bk1) - bk2

    b3p = jnp.pad(b3, (0, coutp - b3.shape[0]))
    h3 = jnp.maximum(out3 + jnp.tile(b3p, NB)[None, :], 0.0)
    pooled = jnp.mean(h3, axis=0).reshape(NB, coutp)[:, :W3.shape[2]]
    return jax.nn.log_softmax(pooled, axis=1)
